# Initial kernel scaffold; baseline (speedup 1.0000x reference)
import functools

import jax
import jax.numpy as jnp
from jax import lax
from jax.experimental import pallas as pl
from jax.experimental.pallas import tpu as pltpu
from jax.experimental.pallas import tpu_sc as plsc

NN = 10000
HH = 128
NG = 128
NC = 2
NS = 16
NW = NC * NS
CHUNK = 128
EF = 2 * 320000
NCH = -(-EF // (NW * CHUNK))
PER_W = NCH * CHUNK
EP = NW * PER_W
NPAD = 10240
RPT = NPAD // NS
ZR = 16

_mesh = functools.partial(
    plsc.VectorSubcoreMesh, core_axis_name="c", subcore_axis_name="s",
    num_cores=NC, num_subcores=NS)


def _wid():
    return lax.axis_index("s") * NC + lax.axis_index("c")


def _codes(row_v, col_v, code_v):
    for k in range(CHUNK // 16):
        sl = pl.ds(k * 16, 16)
        code_v[sl] = row_v[sl] * NN + col_v[sl]


@functools.partial(
    pl.kernel,
    out_type=jax.ShapeDtypeStruct((NN * NN,), jnp.int32),
    mesh=_mesh(),
    scratch_types=[
        pltpu.VMEM((CHUNK,), jnp.int32),
        pltpu.VMEM((CHUNK,), jnp.int32),
        pltpu.VMEM((CHUNK,), jnp.int32),
        pltpu.VMEM((CHUNK,), jnp.int32),
    ],
)
def _dedup_scatter(row_hbm, col_hbm, a_hbm, row_v, col_v, code_v, eid_v):
    base = _wid() * PER_W

    def step(j, carry):
        off = base + j * CHUNK
        pltpu.sync_copy(row_hbm.at[pl.ds(off, CHUNK)], row_v)
        pltpu.sync_copy(col_hbm.at[pl.ds(off, CHUNK)], col_v)
        _codes(row_v, col_v, code_v)
        for k in range(CHUNK // 16):
            sl = pl.ds(k * 16, 16)
            eid_v[sl] = lax.iota(jnp.int32, 16) + (off + k * 16)
        pltpu.sync_copy(eid_v, a_hbm.at[code_v])
        return carry

    lax.fori_loop(0, NCH, step, 0)


@functools.partial(
    pl.kernel,
    out_type=jax.ShapeDtypeStruct((EP,), jnp.int32),
    mesh=_mesh(),
    scratch_types=[
        pltpu.VMEM((CHUNK,), jnp.int32),
        pltpu.VMEM((CHUNK,), jnp.int32),
        pltpu.VMEM((CHUNK,), jnp.int32),
        pltpu.VMEM((CHUNK,), jnp.int32),
        pltpu.VMEM((CHUNK,), jnp.int32),
    ],
)
def _dedup_check(row_hbm, col_hbm, a_hbm, idx_hbm,
                 row_v, col_v, code_v, got_v, out_v):
    base = _wid() * PER_W

    def step(j, carry):
        off = base + j * CHUNK
        pltpu.sync_copy(row_hbm.at[pl.ds(off, CHUNK)], row_v)
        pltpu.sync_copy(col_hbm.at[pl.ds(off, CHUNK)], col_v)
        _codes(row_v, col_v, code_v)
        pltpu.sync_copy(a_hbm.at[code_v], got_v)
        for k in range(CHUNK // 16):
            sl = pl.ds(k * 16, 16)
            eid = lax.iota(jnp.int32, 16) + (off + k * 16)
            keep = (got_v[sl] == eid) & (row_v[sl] != col_v[sl])
            out_v[sl] = jnp.where(keep, row_v[sl], NN)
        pltpu.sync_copy(out_v, idx_hbm.at[pl.ds(off, CHUNK)])
        return carry

    lax.fori_loop(0, NCH, step, 0)


@functools.partial(
    pl.kernel,
    out_type=jax.ShapeDtypeStruct((NC, NPAD, HH), jnp.float32),
    mesh=_mesh(),
    scratch_types=[
        pltpu.VMEM((CHUNK,), jnp.int32),
        pltpu.VMEM((CHUNK,), jnp.int32),
        pltpu.VMEM((CHUNK, HH), jnp.float32),
        pltpu.VMEM((ZR, HH), jnp.float32),
        pltpu.VMEM_SHARED((NPAD, HH), jnp.float32),
        pltpu.SemaphoreType.DMA,
    ],
)
def _sc_aggregate(h_hbm, idx_hbm, col_hbm, out_hbm,
                  idx_v, col_v, rows_v, zero_v, agg_sh, sem):
    cid = lax.axis_index("c")
    sid = lax.axis_index("s")
    base = _wid() * PER_W
    tile_row0 = sid * RPT

    for r in range(ZR):
        for k in range(HH // 16):
            zero_v[r, pl.ds(k * 16, 16)] = jnp.zeros((16,), jnp.float32)

    def zstep(z, carry):
        pltpu.sync_copy(zero_v, agg_sh.at[pl.ds(tile_row0 + z * ZR, ZR)])
        return carry

    lax.fori_loop(0, RPT // ZR, zstep, 0)
    plsc.subcore_barrier()

    def step(j, carry):
        off = base + j * CHUNK
        pltpu.sync_copy(idx_hbm.at[pl.ds(off, CHUNK)], idx_v)
        pltpu.sync_copy(col_hbm.at[pl.ds(off, CHUNK)], col_v)
        pltpu.async_copy(h_hbm.at[col_v], rows_v, sem).wait()
        pltpu.sync_copy(rows_v, agg_sh.at[idx_v], add=True)
        return carry

    lax.fori_loop(0, NCH, step, 0)
    plsc.subcore_barrier()
    pltpu.sync_copy(agg_sh.at[pl.ds(tile_row0, RPT)],
                    out_hbm.at[cid, pl.ds(tile_row0, RPT)])


def _h0_body(x_ref, w_ref, b_ref, o_ref):
    o_ref[...] = (
        jnp.dot(x_ref[...], w_ref[...], preferred_element_type=jnp.float32)
        + b_ref[...])


def _layer_body(h_ref, agg_ref, eps_ref, w1_ref, b1_ref, bng_ref, bnb_ref,
                w2_ref, b2_ref, lng_ref, lnb_ref, o_ref):
    h = h_ref[...]
    agg = agg_ref[0, :NN, :] + agg_ref[1, :NN, :]
    u = (1.0 + eps_ref[0, 0]) * h + agg
    u = jnp.dot(u, w1_ref[...], preferred_element_type=jnp.float32) + b1_ref[...]
    mean = jnp.mean(u, axis=0, keepdims=True)
    var = jnp.mean(jnp.square(u - mean), axis=0, keepdims=True)
    u = (u - mean) / jnp.sqrt(var + 1e-5) * bng_ref[...] + bnb_ref[...]
    u = jnp.maximum(u, 0.0)
    u = jnp.dot(u, w2_ref[...], preferred_element_type=jnp.float32) + b2_ref[...]
    u = jnp.maximum(u, 0.0)
    h = h + u
    mu = jnp.mean(h, axis=1, keepdims=True)
    sig = jnp.sqrt(jnp.mean(jnp.square(h - mu), axis=1, keepdims=True) + 1e-5)
    o_ref[...] = (h - mu) / sig * lng_ref[...] + lnb_ref[...]


def _pool_body(h_ref, b_ref, o_ref):
    gids = lax.broadcasted_iota(jnp.int32, (NN, NG), 1)
    onehot = (b_ref[...] == gids).astype(jnp.float32)
    seg = lax.dot_general(onehot, h_ref[...],
                          dimension_numbers=(((0,), (0,)), ((), ())),
                          preferred_element_type=jnp.float32)
    counts = jnp.sum(onehot, axis=0)[:, None]
    o_ref[...] = seg / jnp.maximum(counts, 1.0)


def kernel(x, edge_index, batch, params):
    f32 = jnp.float32
    src = edge_index[0].astype(jnp.int32)
    dst = edge_index[1].astype(jnp.int32)
    pad = jnp.zeros((EP - EF,), jnp.int32)
    row_p = jnp.concatenate([src, dst, pad])
    col_p = jnp.concatenate([dst, src, pad])

    a_tab = _dedup_scatter(row_p, col_p)
    idx_p = _dedup_check(row_p, col_p, a_tab)

    w0 = params["W0"].astype(f32)
    b0 = params["b0"].astype(f32).reshape(1, HH)
    h = pl.pallas_call(
        _h0_body,
        out_shape=jax.ShapeDtypeStruct((NN, HH), f32),
    )(x.astype(f32), w0, b0)

    for lp in params["layers"]:
        agg2 = _sc_aggregate(h, idx_p, col_p)
        h = pl.pallas_call(
            _layer_body,
            out_shape=jax.ShapeDtypeStruct((NN, HH), f32),
        )(h, agg2,
          lp["eps"].astype(f32).reshape(1, 1),
          lp["W1"].astype(f32), lp["b1"].astype(f32).reshape(1, HH),
          lp["bn_g"].astype(f32).reshape(1, HH),
          lp["bn_b"].astype(f32).reshape(1, HH),
          lp["W2"].astype(f32), lp["b2"].astype(f32).reshape(1, HH),
          lp["ln_g"].astype(f32).reshape(1, HH),
          lp["ln_b"].astype(f32).reshape(1, HH))

    logits = pl.pallas_call(
        _pool_body,
        out_shape=jax.ShapeDtypeStruct((NG, HH), f32),
    )(h, batch.astype(jnp.int32).reshape(NN, 1))
    return logits


# speedup vs baseline: 25.5269x; 25.5269x over previous
import functools

import jax
import jax.numpy as jnp
from jax import lax
from jax.experimental import pallas as pl
from jax.experimental.pallas import tpu as pltpu
from jax.experimental.pallas import tpu_sc as plsc

NN = 10000
HH = 128
NG = 128
NC = 2
NS = 16
NW = NC * NS
CHUNK = 128
EF = 2 * 320000
NCH = -(-EF // (NW * CHUNK))
PER_W = NCH * CHUNK
EP = NW * PER_W
NPAD = 10240
RPT = NPAD // NS
ZR = 16

_mesh = functools.partial(
    plsc.VectorSubcoreMesh, core_axis_name="c", subcore_axis_name="s",
    num_cores=NC, num_subcores=NS)


def _lazy(builder):
    built = []

    def call(*args):
        if not built:
            built.append(builder())
        return built[0](*args)

    return call


def _i32(v):
    return jnp.int32(v)


def _wid():
    return lax.axis_index("s") * _i32(NC) + lax.axis_index("c")


def _codes(row_v, col_v, code_v):
    for k in range(CHUNK // 16):
        sl = pl.ds(k * 16, 16)
        code_v[sl] = row_v[sl] * _i32(NN) + col_v[sl]


@_lazy
def _dedup_scatter():
    @functools.partial(
        pl.kernel,
        out_type=jax.ShapeDtypeStruct((NN * NN,), jnp.int32),
        mesh=_mesh(),
        scratch_types=[
            pltpu.VMEM((CHUNK,), jnp.int32),
            pltpu.VMEM((CHUNK,), jnp.int32),
            pltpu.VMEM((CHUNK,), jnp.int32),
            pltpu.VMEM((CHUNK,), jnp.int32),
        ],
    )
    def body(row_hbm, col_hbm, a_hbm, row_v, col_v, code_v, eid_v):
        base = _wid() * _i32(PER_W)

        def step(j, carry):
            off = base + j * _i32(CHUNK)
            pltpu.sync_copy(row_hbm.at[pl.ds(off, CHUNK)], row_v)
            pltpu.sync_copy(col_hbm.at[pl.ds(off, CHUNK)], col_v)
            _codes(row_v, col_v, code_v)
            for k in range(CHUNK // 16):
                sl = pl.ds(k * 16, 16)
                eid_v[sl] = lax.iota(jnp.int32, 16) + (off + _i32(k * 16))
            pltpu.sync_copy(eid_v, a_hbm.at[code_v])
            return carry

        lax.fori_loop(_i32(0), _i32(NCH), step, _i32(0))

    return body


@_lazy
def _dedup_check():
    @functools.partial(
        pl.kernel,
        out_type=jax.ShapeDtypeStruct((EP,), jnp.int32),
        mesh=_mesh(),
        scratch_types=[
            pltpu.VMEM((CHUNK,), jnp.int32),
            pltpu.VMEM((CHUNK,), jnp.int32),
            pltpu.VMEM((CHUNK,), jnp.int32),
            pltpu.VMEM((CHUNK,), jnp.int32),
            pltpu.VMEM((CHUNK,), jnp.int32),
        ],
    )
    def body(row_hbm, col_hbm, a_hbm, idx_hbm,
             row_v, col_v, code_v, got_v, out_v):
        base = _wid() * _i32(PER_W)

        def step(j, carry):
            off = base + j * _i32(CHUNK)
            pltpu.sync_copy(row_hbm.at[pl.ds(off, CHUNK)], row_v)
            pltpu.sync_copy(col_hbm.at[pl.ds(off, CHUNK)], col_v)
            _codes(row_v, col_v, code_v)
            pltpu.sync_copy(a_hbm.at[code_v], got_v)
            for k in range(CHUNK // 16):
                sl = pl.ds(k * 16, 16)
                eid = lax.iota(jnp.int32, 16) + (off + _i32(k * 16))
                keep = (got_v[sl] == eid) & (row_v[sl] != col_v[sl])
                out_v[sl] = jnp.where(keep, row_v[sl], _i32(NN))
            pltpu.sync_copy(out_v, idx_hbm.at[pl.ds(off, CHUNK)])
            return carry

        lax.fori_loop(_i32(0), _i32(NCH), step, _i32(0))

    return body


@_lazy
def _sc_aggregate():
    @functools.partial(
        pl.kernel,
        out_type=jax.ShapeDtypeStruct((NC, NPAD, HH), jnp.float32),
        mesh=_mesh(),
        scratch_types=[
            pltpu.VMEM((CHUNK,), jnp.int32),
            pltpu.VMEM((CHUNK,), jnp.int32),
            pltpu.VMEM((CHUNK, HH), jnp.float32),
            pltpu.VMEM((ZR, HH), jnp.float32),
            pltpu.VMEM_SHARED((NPAD, HH), jnp.float32),
            pltpu.SemaphoreType.DMA,
        ],
    )
    def body(h_hbm, idx_hbm, col_hbm, out_hbm,
             idx_v, col_v, rows_v, zero_v, agg_sh, sem):
        cid = lax.axis_index("c")
        sid = lax.axis_index("s")
        base = _wid() * _i32(PER_W)
        tile_row0 = sid * _i32(RPT)

        for r in range(ZR):
            for k in range(HH // 16):
                zero_v[r, pl.ds(k * 16, 16)] = jnp.zeros((16,), jnp.float32)

        def zstep(z, carry):
            pltpu.sync_copy(zero_v, agg_sh.at[pl.ds(tile_row0 + z * _i32(ZR), ZR)])
            return carry

        lax.fori_loop(_i32(0), _i32(RPT // ZR), zstep, _i32(0))
        plsc.subcore_barrier()

        def step(j, carry):
            off = base + j * _i32(CHUNK)
            pltpu.sync_copy(idx_hbm.at[pl.ds(off, CHUNK)], idx_v)
            pltpu.sync_copy(col_hbm.at[pl.ds(off, CHUNK)], col_v)
            pltpu.async_copy(h_hbm.at[col_v], rows_v, sem).wait()
            pltpu.sync_copy(rows_v, agg_sh.at[idx_v], add=True)
            return carry

        lax.fori_loop(_i32(0), _i32(NCH), step, _i32(0))
        plsc.subcore_barrier()
        pltpu.sync_copy(agg_sh.at[pl.ds(tile_row0, RPT)],
                        out_hbm.at[cid, pl.ds(tile_row0, RPT)])

    return body


def _h0_body(x_ref, w_ref, b_ref, o_ref):
    o_ref[...] = (
        jnp.dot(x_ref[...], w_ref[...], preferred_element_type=jnp.float32)
        + b_ref[...])


def _layer_body(h_ref, agg_ref, eps_ref, w1_ref, b1_ref, bng_ref, bnb_ref,
                w2_ref, b2_ref, lng_ref, lnb_ref, o_ref):
    h = h_ref[...]
    agg = agg_ref[0, :NN, :] + agg_ref[1, :NN, :]
    u = (1.0 + eps_ref[0, 0]) * h + agg
    u = jnp.dot(u, w1_ref[...], preferred_element_type=jnp.float32) + b1_ref[...]
    mean = jnp.mean(u, axis=0, keepdims=True)
    var = jnp.mean(jnp.square(u - mean), axis=0, keepdims=True)
    u = (u - mean) / jnp.sqrt(var + 1e-5) * bng_ref[...] + bnb_ref[...]
    u = jnp.maximum(u, 0.0)
    u = jnp.dot(u, w2_ref[...], preferred_element_type=jnp.float32) + b2_ref[...]
    u = jnp.maximum(u, 0.0)
    h = h + u
    mu = jnp.mean(h, axis=1, keepdims=True)
    sig = jnp.sqrt(jnp.mean(jnp.square(h - mu), axis=1, keepdims=True) + 1e-5)
    o_ref[...] = (h - mu) / sig * lng_ref[...] + lnb_ref[...]


def _pool_body(h_ref, b_ref, o_ref):
    gids = lax.broadcasted_iota(jnp.int32, (NN, NG), 1)
    onehot = (b_ref[...] == gids).astype(jnp.float32)
    seg = lax.dot_general(onehot, h_ref[...],
                          dimension_numbers=(((0,), (0,)), ((), ())),
                          preferred_element_type=jnp.float32)
    counts = jnp.sum(onehot, axis=0)[:, None]
    o_ref[...] = seg / jnp.maximum(counts, 1.0)


def kernel(x, edge_index, batch, params):
    f32 = jnp.float32
    src = edge_index[0].astype(jnp.int32)
    dst = edge_index[1].astype(jnp.int32)
    pad = jnp.zeros((EP - EF,), jnp.int32)
    row_p = jnp.concatenate([src, dst, pad])
    col_p = jnp.concatenate([dst, src, pad])

    a_tab = _dedup_scatter(row_p, col_p)
    idx_p = _dedup_check(row_p, col_p, a_tab)

    w0 = params["W0"].astype(f32)
    b0 = params["b0"].astype(f32).reshape(1, HH)
    h = pl.pallas_call(
        _h0_body,
        out_shape=jax.ShapeDtypeStruct((NN, HH), f32),
    )(x.astype(f32), w0, b0)

    for lp in params["layers"]:
        agg2 = _sc_aggregate(h, idx_p, col_p)
        h = pl.pallas_call(
            _layer_body,
            out_shape=jax.ShapeDtypeStruct((NN, HH), f32),
        )(h, agg2,
          lp["eps"].astype(f32).reshape(1, 1),
          lp["W1"].astype(f32), lp["b1"].astype(f32).reshape(1, HH),
          lp["bn_g"].astype(f32).reshape(1, HH),
          lp["bn_b"].astype(f32).reshape(1, HH),
          lp["W2"].astype(f32), lp["b2"].astype(f32).reshape(1, HH),
          lp["ln_g"].astype(f32).reshape(1, HH),
          lp["ln_b"].astype(f32).reshape(1, HH))

    logits = pl.pallas_call(
        _pool_body,
        out_shape=jax.ShapeDtypeStruct((NG, HH), f32),
    )(h, batch.astype(jnp.int32).reshape(NN, 1))
    return logits


# speedup vs baseline: 66.9460x; 2.6226x over previous
import functools

import jax
import jax.numpy as jnp
from jax import lax
from jax.experimental import pallas as pl
from jax.experimental.pallas import tpu as pltpu
from jax.experimental.pallas import tpu_sc as plsc

NN = 10000
HH = 128
NG = 128
NC = 2
NS = 16
NW = NC * NS
CHUNK = 128
EF = 2 * 320000
E0 = 320000
KB = 4
NCH = 160
GROUPS = NCH // KB
PER_W = NCH * CHUNK
EP = NW * PER_W
TOT_CH = NW * NCH
NCHD = NCH // 2
PER_WD = NCHD * CHUNK
EPD = NW * PER_WD
TOT_CHD = NW * NCHD
GROUPSD = NCHD // KB
NPAD = 10112
RPT = NPAD // NS
HALF = NCH // 2

_mesh = functools.partial(
    plsc.VectorSubcoreMesh, core_axis_name="c", subcore_axis_name="s",
    num_cores=NC, num_subcores=NS)


def _lazy(builder):
    built = []

    def call(*args):
        if not built:
            built.append(builder())
        return built[0](*args)

    return call


def _i32(v):
    return jnp.int32(v)


def _wid():
    return lax.axis_index("s") * _i32(NC) + lax.axis_index("c")


@_lazy
def _dedup_scatter():
    @functools.partial(
        pl.kernel,
        out_type=jax.ShapeDtypeStruct((NN * NN,), jnp.int32),
        mesh=_mesh(),
        scratch_types=(
            [pltpu.VMEM((NCHD, CHUNK), jnp.int32)] * 2
            + [pltpu.VMEM((CHUNK,), jnp.int32)] * (2 * KB)
            + [pltpu.SemaphoreType.DMA] * KB
        ),
    )
    def body(row_hbm, col_hbm, a_hbm, row_blk, col_blk, *rest):
        code_b = rest[0:KB]
        eid_b = rest[KB:2 * KB]
        sems = rest[2 * KB:3 * KB]
        ch0 = pl.multiple_of(_wid() * _i32(NCHD), 8)
        pltpu.sync_copy(row_hbm.at[pl.ds(ch0, NCHD)], row_blk)
        pltpu.sync_copy(col_hbm.at[pl.ds(ch0, NCHD)], col_blk)

        def step(g, carry):
            j0 = g * _i32(KB)
            for b in range(KB):
                j = j0 + _i32(b)

                @pl.when(g > 0)
                def _wait():
                    pltpu.make_async_copy(
                        eid_b[b], a_hbm.at[code_b[b]], sems[b]).wait()

                off = (ch0 + j) * _i32(CHUNK)
                for k in range(CHUNK // 16):
                    sl = pl.ds(k * 16, 16)
                    lo = jnp.minimum(row_blk[j, sl], col_blk[j, sl])
                    hi = jnp.maximum(row_blk[j, sl], col_blk[j, sl])
                    code_b[b][sl] = lo * _i32(NN) + hi
                    eid_b[b][sl] = lax.iota(jnp.int32, 16) + (off + _i32(k * 16))
                pltpu.async_copy(eid_b[b], a_hbm.at[code_b[b]], sems[b])
            return carry

        lax.fori_loop(_i32(0), _i32(GROUPSD), step, _i32(0))
        for b in range(KB):
            pltpu.make_async_copy(eid_b[b], a_hbm.at[code_b[b]], sems[b]).wait()

    return body


@_lazy
def _dedup_check():
    @functools.partial(
        pl.kernel,
        out_type=jax.ShapeDtypeStruct((TOT_CH, CHUNK), jnp.int32),
        mesh=_mesh(),
        scratch_types=(
            [pltpu.VMEM((NCHD, CHUNK), jnp.int32)] * 2
            + [pltpu.VMEM((CHUNK,), jnp.int32)] * (4 * KB)
            + [pltpu.SemaphoreType.DMA] * (3 * KB)
        ),
    )
    def body(row_hbm, col_hbm, a_hbm, idx_hbm, row_blk, col_blk, *rest):
        code_b = rest[0:KB]
        got_b = rest[KB:2 * KB]
        out_b = rest[2 * KB:3 * KB]
        out2_b = rest[3 * KB:4 * KB]
        gsem = rest[4 * KB:5 * KB]
        ssem = rest[5 * KB:6 * KB]
        s2sem = rest[6 * KB:7 * KB]
        ch0 = pl.multiple_of(_wid() * _i32(NCHD), 8)
        pltpu.sync_copy(row_hbm.at[pl.ds(ch0, NCHD)], row_blk)
        pltpu.sync_copy(col_hbm.at[pl.ds(ch0, NCHD)], col_blk)

        def step(g, carry):
            j0 = g * _i32(KB)
            gd = []
            for b in range(KB):
                j = j0 + _i32(b)

                @pl.when(g > 0)
                def _wait():
                    pltpu.make_async_copy(
                        out_b[b], idx_hbm.at[_i32(0)], ssem[b]).wait()
                    pltpu.make_async_copy(
                        out2_b[b], idx_hbm.at[_i32(0)], s2sem[b]).wait()

                for k in range(CHUNK // 16):
                    sl = pl.ds(k * 16, 16)
                    lo = jnp.minimum(row_blk[j, sl], col_blk[j, sl])
                    hi = jnp.maximum(row_blk[j, sl], col_blk[j, sl])
                    code_b[b][sl] = lo * _i32(NN) + hi
                gd.append(pltpu.async_copy(a_hbm.at[code_b[b]], got_b[b],
                                           gsem[b]))
            for b in range(KB):
                j = j0 + _i32(b)
                off = (ch0 + j) * _i32(CHUNK)
                dump = _i32(NN) + lax.rem(ch0 + j, _i32(96))
                gd[b].wait()
                for k in range(CHUNK // 16):
                    sl = pl.ds(k * 16, 16)
                    eid = lax.iota(jnp.int32, 16) + (off + _i32(k * 16))
                    r = row_blk[j, sl]
                    c = col_blk[j, sl]
                    keep = (got_b[b][sl] == eid) & (r != c)
                    self = jnp.where(keep, r, dump)
                    selr = jnp.where(keep, c, dump)
                    out_b[b][sl] = self * _i32(65536) + c
                    out2_b[b][sl] = selr * _i32(65536) + r
                pltpu.async_copy(out_b[b], idx_hbm.at[ch0 + j], ssem[b])
                pltpu.async_copy(out2_b[b],
                                 idx_hbm.at[_i32(TOT_CHD) + ch0 + j], s2sem[b])
            return carry

        lax.fori_loop(_i32(0), _i32(GROUPSD), step, _i32(0))
        for b in range(KB):
            pltpu.make_async_copy(out_b[b], idx_hbm.at[_i32(0)], ssem[b]).wait()
            pltpu.make_async_copy(out2_b[b], idx_hbm.at[_i32(0)],
                                  s2sem[b]).wait()

    return body


@_lazy
def _sc_aggregate():
    @functools.partial(
        pl.kernel,
        out_type=jax.ShapeDtypeStruct((NC, NPAD, HH), jnp.float32),
        mesh=_mesh(),
        scratch_types=(
            [pltpu.VMEM((HALF, CHUNK), jnp.int32)]
            + [pltpu.VMEM((CHUNK, HH), jnp.float32)] * 2
            + [pltpu.VMEM((CHUNK,), jnp.int32)] * 8
            + [pltpu.VMEM_SHARED((NPAD, HH), jnp.float32)]
            + [pltpu.SemaphoreType.DMA] * 5
        ),
    )
    def body(h_hbm, pk_hbm, out_hbm, slab, r0, r1, *rest):
        ibuf = rest[0:4]
        cbuf = rest[4:8]
        agg_sh = rest[8]
        gsem = rest[9:11]
        ssem = rest[11:13]
        zsem = rest[13]
        rows = (r0, r1)
        cid = lax.axis_index("c")
        sid = lax.axis_index("s")
        ch0 = pl.multiple_of(_wid() * _i32(NCH), 8)
        tile_row0 = pl.multiple_of(sid * _i32(RPT), 8)

        for r in range(CHUNK):
            for k in range(HH // 16):
                r0[r, pl.ds(k * 16, 16)] = jnp.zeros((16,), jnp.float32)
        zd = []
        nfull, rem = RPT // CHUNK, RPT % CHUNK
        for z in range(nfull):
            zd.append(pltpu.async_copy(
                r0, agg_sh.at[pl.ds(tile_row0 + _i32(z * CHUNK), CHUNK)],
                zsem))
        if rem:
            zd.append(pltpu.async_copy(
                r0.at[pl.ds(0, rem)],
                agg_sh.at[pl.ds(tile_row0 + _i32(nfull * CHUNK), rem)], zsem))
        pltpu.sync_copy(pk_hbm.at[pl.ds(ch0, HALF)], slab)
        for d in zd:
            d.wait()
        plsc.subcore_barrier()

        def step(g, carry):
            t0 = g * _i32(4)
            for c in range(4):
                t = t0 + _i32(c)
                cp = (c + 3) % 4
                rp = (c + 1) % 2

                def bstep():
                    pltpu.make_async_copy(h_hbm.at[cbuf[cp]], rows[rp],
                                          gsem[rp]).wait()
                    pltpu.async_copy(rows[rp], agg_sh.at[ibuf[cp]],
                                     ssem[rp], add=True)

                if c == 0:
                    pl.when(g > 0)(bstep)
                else:
                    bstep()

                def wait_scat():
                    pltpu.make_async_copy(rows[c % 2],
                                          agg_sh.at[ibuf[c % 4]],
                                          ssem[c % 2]).wait()

                if c < 2:
                    pl.when(g > 0)(wait_scat)
                else:
                    wait_scat()

                if c == 0:
                    @pl.when(g == _i32(GROUPS // 2))
                    def _reload():
                        pltpu.sync_copy(
                            pk_hbm.at[pl.ds(ch0 + _i32(HALF), HALF)], slab)

                jl = jnp.where(t < _i32(HALF), t, t - _i32(HALF))
                for k in range(CHUNK // 16):
                    sl = pl.ds(k * 16, 16)
                    pk = slab[jl, sl]
                    ibuf[c][sl] = lax.shift_right_arithmetic(pk, _i32(16))
                    cbuf[c][sl] = pk & _i32(0xFFFF)
                pltpu.async_copy(h_hbm.at[cbuf[c]], rows[c % 2], gsem[c % 2])
            return carry

        lax.fori_loop(_i32(0), _i32(GROUPS), step, _i32(0))
        pltpu.make_async_copy(h_hbm.at[cbuf[3]], rows[1], gsem[1]).wait()
        pltpu.async_copy(rows[1], agg_sh.at[ibuf[3]], ssem[1], add=True)
        for rbi in range(2):
            pltpu.make_async_copy(rows[rbi], agg_sh.at[ibuf[0]],
                                  ssem[rbi]).wait()
        plsc.subcore_barrier()
        pltpu.sync_copy(agg_sh.at[pl.ds(tile_row0, RPT)],
                        out_hbm.at[cid, pl.ds(tile_row0, RPT)])

    return body


def _h0_body(x_ref, w_ref, b_ref, o_ref):
    o_ref[...] = (
        jnp.dot(x_ref[...], w_ref[...], preferred_element_type=jnp.float32)
        + b_ref[...])


def _layer_body(h_ref, agg_ref, eps_ref, w1_ref, b1_ref, bng_ref, bnb_ref,
                w2_ref, b2_ref, lng_ref, lnb_ref, o_ref):
    h = h_ref[...]
    agg = agg_ref[0, :NN, :] + agg_ref[1, :NN, :]
    u = (1.0 + eps_ref[0, 0]) * h + agg
    u = jnp.dot(u, w1_ref[...], preferred_element_type=jnp.float32) + b1_ref[...]
    mean = jnp.mean(u, axis=0, keepdims=True)
    var = jnp.mean(jnp.square(u - mean), axis=0, keepdims=True)
    u = (u - mean) / jnp.sqrt(var + 1e-5) * bng_ref[...] + bnb_ref[...]
    u = jnp.maximum(u, 0.0)
    u = jnp.dot(u, w2_ref[...], preferred_element_type=jnp.float32) + b2_ref[...]
    u = jnp.maximum(u, 0.0)
    h = h + u
    mu = jnp.mean(h, axis=1, keepdims=True)
    sig = jnp.sqrt(jnp.mean(jnp.square(h - mu), axis=1, keepdims=True) + 1e-5)
    o_ref[...] = (h - mu) / sig * lng_ref[...] + lnb_ref[...]


def _pool_body(h_ref, b_ref, o_ref):
    gids = lax.broadcasted_iota(jnp.int32, (NN, NG), 1)
    onehot = (b_ref[...] == gids).astype(jnp.float32)
    seg = lax.dot_general(onehot, h_ref[...],
                          dimension_numbers=(((0,), (0,)), ((), ())),
                          preferred_element_type=jnp.float32)
    counts = jnp.sum(onehot, axis=0)[:, None]
    o_ref[...] = seg / jnp.maximum(counts, 1.0)


def kernel(x, edge_index, batch, params):
    f32 = jnp.float32
    src = edge_index[0].astype(jnp.int32)
    dst = edge_index[1].astype(jnp.int32)
    pad = (jnp.arange(EPD - E0, dtype=jnp.int32) % NN)
    row_p = jnp.concatenate([src, pad]).reshape(TOT_CHD, CHUNK)
    col_p = jnp.concatenate([dst, pad]).reshape(TOT_CHD, CHUNK)

    a_tab = _dedup_scatter(row_p, col_p)
    pk_p = _dedup_check(row_p, col_p, a_tab)

    w0 = params["W0"].astype(f32)
    b0 = params["b0"].astype(f32).reshape(1, HH)
    h = pl.pallas_call(
        _h0_body,
        out_shape=jax.ShapeDtypeStruct((NN, HH), f32),
    )(x.astype(f32), w0, b0)

    for lp in params["layers"]:
        agg2 = _sc_aggregate(h, pk_p)
        h = pl.pallas_call(
            _layer_body,
            out_shape=jax.ShapeDtypeStruct((NN, HH), f32),
        )(h, agg2,
          lp["eps"].astype(f32).reshape(1, 1),
          lp["W1"].astype(f32), lp["b1"].astype(f32).reshape(1, HH),
          lp["bn_g"].astype(f32).reshape(1, HH),
          lp["bn_b"].astype(f32).reshape(1, HH),
          lp["W2"].astype(f32), lp["b2"].astype(f32).reshape(1, HH),
          lp["ln_g"].astype(f32).reshape(1, HH),
          lp["ln_b"].astype(f32).reshape(1, HH))

    logits = pl.pallas_call(
        _pool_body,
        out_shape=jax.ShapeDtypeStruct((NG, HH), f32),
    )(h, batch.astype(jnp.int32).reshape(NN, 1))
    return logits
